# baseline (speedup 1.0000x reference)
import functools

import jax
import jax.numpy as jnp
from jax import lax
from jax.experimental import pallas as pl
from jax.experimental.pallas import tpu as pltpu
from jax.experimental.pallas import tpu_sc as plsc

_NB_KNOTS = 128
_X_MIN = -3.0
_X_MAX = 3.0
_EPS = 1e-06

_NC = 2
_NS = 16
_NW = _NC * _NS
_L = 16

_ROWS = 2048
_COLS = 4096
_RPW = _ROWS // _NW
_CR = 8
_CC = _COLS // 4
_NBUF = 4
_NBLK = (_RPW // _CR) * (_COLS // _CC)
_NSS = _NBLK // _NBUF


def _build_tables(theta_ref, y_ref, dy_ref):
    lane = lax.iota(jnp.int32, _L)
    zero = jnp.zeros((_L,), jnp.float32)
    carry = zero
    for k in range(_NB_KNOTS // _L):
        v = theta_ref[pl.ds(k * _L, _L)]
        d = jnp.exp(v) + jnp.float32(_EPS)
        if k == 0:
            d = jnp.where(lane == 0, v, d)
        c = d
        for s in (1, 2, 4, 8):
            y_ref[pl.ds(k * _L, _L)] = c
            shifted = plsc.load_gather(
                y_ref, [jnp.maximum(lane - s, 0) + k * _L])
            c = c + jnp.where(lane >= s, shifted, zero)
        c = c + carry
        y_ref[pl.ds(k * _L, _L)] = c
        carry = plsc.load_gather(
            y_ref, [jnp.full((_L,), k * _L + _L - 1, jnp.int32)]
        )
    for k in range(_NB_KNOTS // _L):
        idx = lane + k * _L
        yl = plsc.load_gather(y_ref, [idx])
        yr = plsc.load_gather(y_ref, [jnp.minimum(idx + 1, _NB_KNOTS - 1)])
        dy_ref[pl.ds(k * _L, _L)] = yr - yl


def _interp_block(inb, outb, y_ref, dy_ref):
    scale = jnp.float32((_NB_KNOTS - 1) / (_X_MAX - _X_MIN))
    for r in range(_CR):
        @plsc.parallel_loop(0, _CC, step=_L, unroll=4)
        def body(off):
            zv = inb[r, pl.ds(off, _L)]
            zn = (zv - jnp.float32(_X_MIN)) * scale
            znc = jnp.minimum(jnp.maximum(zn, jnp.float32(0.0)),
                              jnp.float32(_NB_KNOTS - 2))
            ii = znc.astype(jnp.int32)
            t = zn - ii.astype(jnp.float32)
            yl = plsc.load_gather(y_ref, [ii])
            dy = plsc.load_gather(dy_ref, [ii])
            outb[r, pl.ds(off, _L)] = yl + t * dy


@functools.partial(
    pl.kernel,
    mesh=plsc.VectorSubcoreMesh(core_axis_name="c", subcore_axis_name="s"),
    out_type=jax.ShapeDtypeStruct((_ROWS, _COLS), jnp.float32),
    compiler_params=pltpu.CompilerParams(needs_layout_passes=False),
    scratch_types=[
        pltpu.VMEM((_NB_KNOTS,), jnp.float32),
        pltpu.VMEM((_NB_KNOTS,), jnp.float32),
        pltpu.VMEM((_NB_KNOTS,), jnp.float32),
        pltpu.VMEM((_CR, _CC), jnp.float32),
        pltpu.VMEM((_CR, _CC), jnp.float32),
        pltpu.VMEM((_CR, _CC), jnp.float32),
        pltpu.VMEM((_CR, _CC), jnp.float32),
        pltpu.VMEM((_CR, _CC), jnp.float32),
        pltpu.VMEM((_CR, _CC), jnp.float32),
        pltpu.VMEM((_CR, _CC), jnp.float32),
        pltpu.VMEM((_CR, _CC), jnp.float32),
        pltpu.SemaphoreType.DMA,
        pltpu.SemaphoreType.DMA,
        pltpu.SemaphoreType.DMA,
        pltpu.SemaphoreType.DMA,
        pltpu.SemaphoreType.DMA,
        pltpu.SemaphoreType.DMA,
        pltpu.SemaphoreType.DMA,
        pltpu.SemaphoreType.DMA,
    ],
)
def _spline_sc(z_hbm, theta_hbm, out_hbm,
               theta_v, y_v, dy_v,
               ib0, ib1, ib2, ib3, ob0, ob1, ob2, ob3,
               si0, si1, si2, si3, so0, so1, so2, so3):
    wid = lax.axis_index("s") * _NC + lax.axis_index("c")
    row0 = wid * _RPW
    ibs = (ib0, ib1, ib2, ib3)
    obs = (ob0, ob1, ob2, ob3)
    sis = (si0, si1, si2, si3)
    sos = (so0, so1, so2, so3)

    pltpu.sync_copy(theta_hbm, theta_v)
    _build_tables(theta_v, y_v, dy_v)

    _NQ = _COLS // _CC

    def blk_slice(ref, c):
        g = c // _NQ
        q = lax.rem(c, _NQ)
        return ref.at[pl.ds(row0 + g * _CR, _CR), pl.ds(q * _CC, _CC)]

    for b in range(_NBUF):
        pltpu.async_copy(blk_slice(z_hbm, jnp.int32(b)), ibs[b], sis[b])

    def superstep(s, carry):
        for b in range(_NBUF):
            c = s * _NBUF + b
            pltpu.make_async_copy(blk_slice(z_hbm, c), ibs[b], sis[b]).wait()

            @pl.when(s > 0)
            def _():
                pltpu.make_async_copy(obs[b], blk_slice(out_hbm, c), sos[b]).wait()

            _interp_block(ibs[b], obs[b], y_v, dy_v)
            pltpu.async_copy(obs[b], blk_slice(out_hbm, c), sos[b])

            @pl.when(s + 1 < _NSS)
            def _():
                pltpu.async_copy(blk_slice(z_hbm, c + _NBUF), ibs[b], sis[b])
        return carry

    lax.fori_loop(0, _NSS, superstep, None)
    for b in range(_NBUF):
        c = (_NSS - 1) * _NBUF + b
        pltpu.make_async_copy(obs[b], blk_slice(out_hbm, jnp.int32(c)), sos[b]).wait()


def kernel(z, theta):
    return _spline_sc(z, theta)


# speedup vs baseline: 1.0577x; 1.0577x over previous
import functools

import jax
import jax.numpy as jnp
from jax import lax
from jax.experimental import pallas as pl
from jax.experimental.pallas import tpu as pltpu
from jax.experimental.pallas import tpu_sc as plsc

_NB_KNOTS = 128
_X_MIN = -3.0
_X_MAX = 3.0
_EPS = 1e-06

_NC = 2
_NS = 16
_NW = _NC * _NS
_L = 16

_ROWS = 2048
_COLS = 4096
_RPW = _ROWS // _NW
_CR = 8
_CC = _COLS // 2
_NBUF = 3
_NBLK = (_RPW // _CR) * (_COLS // _CC)
_NSS = _NBLK // _NBUF


def _build_tables(theta_ref, y_ref, dy_ref):
    lane = lax.iota(jnp.int32, _L)
    zero = jnp.zeros((_L,), jnp.float32)
    carry = zero
    for k in range(_NB_KNOTS // _L):
        v = theta_ref[pl.ds(k * _L, _L)]
        d = jnp.exp(v) + jnp.float32(_EPS)
        if k == 0:
            d = jnp.where(lane == 0, v, d)
        c = d
        for s in (1, 2, 4, 8):
            y_ref[pl.ds(k * _L, _L)] = c
            shifted = plsc.load_gather(
                y_ref, [jnp.maximum(lane - s, 0) + k * _L])
            c = c + jnp.where(lane >= s, shifted, zero)
        c = c + carry
        y_ref[pl.ds(k * _L, _L)] = c
        carry = plsc.load_gather(
            y_ref, [jnp.full((_L,), k * _L + _L - 1, jnp.int32)]
        )
    for k in range(_NB_KNOTS // _L):
        idx = lane + k * _L
        yl = plsc.load_gather(y_ref, [idx])
        yr = plsc.load_gather(y_ref, [jnp.minimum(idx + 1, _NB_KNOTS - 1)])
        dy_ref[pl.ds(k * _L, _L)] = yr - yl


def _interp_block(inb, outb, y_ref, dy_ref):
    scale = jnp.float32((_NB_KNOTS - 1) / (_X_MAX - _X_MIN))
    for r in range(_CR):
        @plsc.parallel_loop(0, _CC, step=_L, unroll=4)
        def body(off):
            zv = inb[r, pl.ds(off, _L)]
            zn = (zv - jnp.float32(_X_MIN)) * scale
            znc = jnp.minimum(jnp.maximum(zn, jnp.float32(0.0)),
                              jnp.float32(_NB_KNOTS - 2))
            ii = znc.astype(jnp.int32)
            t = zn - ii.astype(jnp.float32)
            yl = plsc.load_gather(y_ref, [ii])
            dy = plsc.load_gather(dy_ref, [ii])
            outb[r, pl.ds(off, _L)] = yl + t * dy


@functools.partial(
    pl.kernel,
    mesh=plsc.VectorSubcoreMesh(core_axis_name="c", subcore_axis_name="s"),
    out_type=jax.ShapeDtypeStruct((_ROWS, _COLS), jnp.float32),
    compiler_params=pltpu.CompilerParams(needs_layout_passes=False),
    scratch_types=[
        pltpu.VMEM((_NB_KNOTS,), jnp.float32),
        pltpu.VMEM((_NB_KNOTS,), jnp.float32),
        pltpu.VMEM((_NB_KNOTS,), jnp.float32),
        pltpu.VMEM((_CR, _CC), jnp.float32),
        pltpu.VMEM((_CR, _CC), jnp.float32),
        pltpu.VMEM((_CR, _CC), jnp.float32),
        pltpu.VMEM((_CR, _CC), jnp.float32),
        pltpu.VMEM((_CR, _CC), jnp.float32),
        pltpu.VMEM((_CR, _CC), jnp.float32),
        pltpu.SemaphoreType.DMA,
        pltpu.SemaphoreType.DMA,
        pltpu.SemaphoreType.DMA,
        pltpu.SemaphoreType.DMA,
        pltpu.SemaphoreType.DMA,
        pltpu.SemaphoreType.DMA,
    ],
)
def _spline_sc(z_hbm, theta_hbm, out_hbm,
               theta_v, y_v, dy_v,
               ib0, ib1, ib2, ob0, ob1, ob2,
               si0, si1, si2, so0, so1, so2):
    wid = lax.axis_index("s") * _NC + lax.axis_index("c")
    row0 = wid * _RPW
    ibs = (ib0, ib1, ib2)
    obs = (ob0, ob1, ob2)
    sis = (si0, si1, si2)
    sos = (so0, so1, so2)

    pltpu.sync_copy(theta_hbm, theta_v)
    _build_tables(theta_v, y_v, dy_v)

    _NQ = _COLS // _CC

    def blk_slice(ref, c):
        g = c // _NQ
        q = lax.rem(c, _NQ)
        return ref.at[pl.ds(row0 + g * _CR, _CR), pl.ds(q * _CC, _CC)]

    for b in range(_NBUF):
        pltpu.async_copy(blk_slice(z_hbm, jnp.int32(b)), ibs[b], sis[b])

    def section(c, b):
        pltpu.make_async_copy(blk_slice(z_hbm, c), ibs[b], sis[b]).wait()

        @pl.when(c >= _NBUF)
        def _():
            pltpu.make_async_copy(obs[b], blk_slice(out_hbm, c), sos[b]).wait()

        _interp_block(ibs[b], obs[b], y_v, dy_v)
        pltpu.async_copy(obs[b], blk_slice(out_hbm, c), sos[b])

        @pl.when(c + _NBUF < _NBLK)
        def _():
            pltpu.async_copy(blk_slice(z_hbm, c + _NBUF), ibs[b], sis[b])

    def superstep(s, carry):
        for b in range(_NBUF):
            section(s * _NBUF + b, b)
        return carry

    lax.fori_loop(0, _NSS, superstep, None)
    section(jnp.int32(_NSS * _NBUF), 0)
    for b, c in ((1, _NBLK - 3), (2, _NBLK - 2), (0, _NBLK - 1)):
        pltpu.make_async_copy(obs[b], blk_slice(out_hbm, jnp.int32(c)), sos[b]).wait()


def kernel(z, theta):
    return _spline_sc(z, theta)
